# baseline (speedup 1.0000x reference)
import functools

import jax
import jax.numpy as jnp
from jax import lax
from jax.experimental import pallas as pl
from jax.experimental.pallas import tpu as pltpu
from jax.experimental.pallas import tpu_sc as plsc

MEM_LENGTH = 8192
HIDDEN = 768
BATCH = 4

_NC = 2
_NS = 16
_L = 16
_NW = _NC * _NS
_LPW = MEM_LENGTH // _NW
_C = 16
_NJ = HIDDEN // _L
_JB = 8

_GATHER_DNUMS = lax.GatherDimensionNumbers(
    offset_dims=(), collapsed_slice_dims=(0,), start_index_map=(0,))


def _lane_shuffle(v, perm):
    return lax.gather(v, perm[:, None], _GATHER_DNUMS, slice_sizes=(1,),
                      mode=lax.GatherScatterMode.PROMISE_IN_BOUNDS)


def _allsum_vec(v):
    idx = lax.iota(jnp.int32, 16)
    for sh in (8, 4, 2, 1):
        perm = lax.bitwise_xor(idx, jnp.int32(sh))
        v = v + _lane_shuffle(v, perm)
    return v


def _rsqrt_vec(x):
    i = lax.bitcast_convert_type(x, jnp.int32)
    i = jnp.int32(0x5F3759DF) - lax.shift_right_logical(i, 1)
    y = lax.bitcast_convert_type(i, jnp.float32)
    for _ in range(3):
        y = y * (1.5 - 0.5 * x * y * y)
    return y


def _sc_body(mem, emb, gamma, beta, out, gamma_v, beta_v, idx_v, x_v, rs_v, ms_v):
    cid = lax.axis_index("c")
    sid = lax.axis_index("s")
    wid = sid * _NC + cid
    pltpu.sync_copy(gamma, gamma_v)
    pltpu.sync_copy(beta, beta_v)
    base = wid * _LPW

    zero = jnp.zeros((_L,), jnp.float32)
    lane_iota = lax.iota(jnp.int32, _L)

    def chunk(i, carry):
        l0 = base + i * _C
        idx_v[...] = lane_iota + l0
        for b in range(BATCH):
            pltpu.sync_copy(mem.at[b, pl.ds(l0, _C)], x_v)
            pltpu.sync_copy(emb.at[idx_v], x_v, add=True)

            def pass1(r, rc):
                s = zero
                sq = zero
                for j in range(_NJ):
                    sl = pl.ds(j * _L, _L)
                    v = x_v[r, sl]
                    s = s + v
                    sq = sq + v * v
                mean = _allsum_vec(s) * (1.0 / HIDDEN)
                var = _allsum_vec(sq) * (1.0 / HIDDEN) - mean * mean
                rstd = _rsqrt_vec(var + 1e-5)
                rs_v[r, :] = rstd
                ms_v[r, :] = mean * rstd
                return rc

            lax.fori_loop(0, _C, pass1, 0)

            for jb in range(_NJ // _JB):
                gs = [gamma_v[pl.ds((jb * _JB + k) * _L, _L)] for k in range(_JB)]
                bs = [beta_v[pl.ds((jb * _JB + k) * _L, _L)] for k in range(_JB)]

                def pass2(r, rc, jb=jb, gs=gs, bs=bs):
                    rs = rs_v[r, :]
                    ms = ms_v[r, :]
                    for k in range(_JB):
                        sl = pl.ds((jb * _JB + k) * _L, _L)
                        v = x_v[r, sl]
                        x_v[r, sl] = (v * rs - ms) * gs[k] + bs[k]
                    return rc

                lax.fori_loop(0, _C, pass2, 0)

            pltpu.sync_copy(x_v, out.at[b, pl.ds(l0, _C)])
        return carry

    lax.fori_loop(0, _LPW // _C, chunk, 0)


@jax.jit
def kernel(mem, emb_table, gamma, beta):
    mesh = plsc.VectorSubcoreMesh(core_axis_name="c", subcore_axis_name="s")
    run = pl.kernel(
        _sc_body,
        mesh=mesh,
        out_type=jax.ShapeDtypeStruct((BATCH, MEM_LENGTH, HIDDEN), jnp.float32),
        scratch_types=[
            pltpu.VMEM((HIDDEN,), jnp.float32),
            pltpu.VMEM((HIDDEN,), jnp.float32),
            pltpu.VMEM((_L,), jnp.int32),
            pltpu.VMEM((_C, HIDDEN), jnp.float32),
            pltpu.VMEM((_C, _L), jnp.float32),
            pltpu.VMEM((_C, _L), jnp.float32),
        ],
    )
    return run(mem, emb_table, gamma, beta)


# speedup vs baseline: 1.5950x; 1.5950x over previous
import functools

import jax
import jax.numpy as jnp
from jax import lax
from jax.experimental import pallas as pl
from jax.experimental.pallas import tpu as pltpu
from jax.experimental.pallas import tpu_sc as plsc

MEM_LENGTH = 8192
HIDDEN = 768
BATCH = 4

_BL = 512

_NC = 2
_NS = 16
_L = 16
_NW = _NC * _NS
_LPW = MEM_LENGTH // _NW
_C = 16
_NJ = HIDDEN // _L
_JB = 8

_GATHER_DNUMS = lax.GatherDimensionNumbers(
    offset_dims=(), collapsed_slice_dims=(0,), start_index_map=(0,))


def _lane_shuffle(v, perm):
    return lax.gather(v, perm[:, None], _GATHER_DNUMS, slice_sizes=(1,),
                      mode=lax.GatherScatterMode.PROMISE_IN_BOUNDS)


def _allsum_vec(v):
    idx = lax.iota(jnp.int32, 16)
    for sh in (8, 4, 2, 1):
        perm = lax.bitwise_xor(idx, jnp.int32(sh))
        v = v + _lane_shuffle(v, perm)
    return v


def _rsqrt_vec(x):
    i = lax.bitcast_convert_type(x, jnp.int32)
    i = jnp.int32(0x5F3759DF) - lax.shift_right_logical(i, 1)
    y = lax.bitcast_convert_type(i, jnp.float32)
    for _ in range(3):
        y = y * (1.5 - 0.5 * x * y * y)
    return y


_TCB = 3


def _ln_body(mem_ref, emb_ref, gamma_ref, beta_ref, out_ref):
    x = mem_ref[0] + emb_ref[...]
    mean = jnp.mean(x, axis=-1, keepdims=True)
    xc = x - mean
    var = jnp.mean(xc * xc, axis=-1, keepdims=True)
    inv = jax.lax.rsqrt(var + 1e-5)
    out_ref[0] = xc * inv * gamma_ref[...] + beta_ref[...]


def _tc_part(mem, emb_table, gamma, beta):
    nb = mem.shape[0]
    grid = (nb, MEM_LENGTH // _BL)
    return pl.pallas_call(
        _ln_body,
        grid=grid,
        in_specs=[
            pl.BlockSpec((1, _BL, HIDDEN), lambda b, i: (b, i, 0)),
            pl.BlockSpec((_BL, HIDDEN), lambda b, i: (i, 0)),
            pl.BlockSpec((HIDDEN,), lambda b, i: (0,)),
            pl.BlockSpec((HIDDEN,), lambda b, i: (0,)),
        ],
        out_specs=pl.BlockSpec((1, _BL, HIDDEN), lambda b, i: (b, i, 0)),
        out_shape=jax.ShapeDtypeStruct((nb, MEM_LENGTH, HIDDEN), jnp.float32),
    )(mem, emb_table, gamma, beta)


def _sc_body(nb, mem, emb, gamma, beta, out, gamma_v, beta_v, emb_v, x_v, rs_v, ms_v):
    cid = lax.axis_index("c")
    sid = lax.axis_index("s")
    wid = sid * _NC + cid
    pltpu.sync_copy(gamma, gamma_v)
    pltpu.sync_copy(beta, beta_v)
    base = wid * _LPW

    zero = jnp.zeros((_L,), jnp.float32)

    def chunk(i, carry):
        l0 = base + i * _C
        pltpu.sync_copy(emb.at[pl.ds(l0, _C)], emb_v)
        for b in range(nb):
            pltpu.sync_copy(mem.at[b, pl.ds(l0, _C)], x_v)

            def pass1(r, rc):
                s = zero
                sq = zero
                for j in range(_NJ):
                    sl = pl.ds(j * _L, _L)
                    v = x_v[r, sl] + emb_v[r, sl]
                    x_v[r, sl] = v
                    s = s + v
                    sq = sq + v * v
                mean = _allsum_vec(s) * (1.0 / HIDDEN)
                var = _allsum_vec(sq) * (1.0 / HIDDEN) - mean * mean
                rstd = _rsqrt_vec(var + 1e-5)
                rs_v[r, :] = rstd
                ms_v[r, :] = mean * rstd
                return rc

            lax.fori_loop(0, _C, pass1, 0)

            for jb in range(_NJ // _JB):
                gs = [gamma_v[pl.ds((jb * _JB + k) * _L, _L)] for k in range(_JB)]
                bs = [beta_v[pl.ds((jb * _JB + k) * _L, _L)] for k in range(_JB)]

                def pass2(r, rc, jb=jb, gs=gs, bs=bs):
                    rs = rs_v[r, :]
                    ms = ms_v[r, :]
                    for k in range(_JB):
                        sl = pl.ds((jb * _JB + k) * _L, _L)
                        v = x_v[r, sl]
                        x_v[r, sl] = (v * rs - ms) * gs[k] + bs[k]
                    return rc

                lax.fori_loop(0, _C, pass2, 0)

            pltpu.sync_copy(x_v, out.at[b, pl.ds(l0, _C)])
        return carry

    lax.fori_loop(0, _LPW // _C, chunk, 0)


def _sc_part(mem, emb_table, gamma, beta):
    nb = mem.shape[0]
    mesh = plsc.VectorSubcoreMesh(core_axis_name="c", subcore_axis_name="s")
    run = pl.kernel(
        functools.partial(_sc_body, nb),
        mesh=mesh,
        out_type=jax.ShapeDtypeStruct((nb, MEM_LENGTH, HIDDEN), jnp.float32),
        scratch_types=[
            pltpu.VMEM((HIDDEN,), jnp.float32),
            pltpu.VMEM((HIDDEN,), jnp.float32),
            pltpu.VMEM((_C, HIDDEN), jnp.float32),
            pltpu.VMEM((_C, HIDDEN), jnp.float32),
            pltpu.VMEM((_C, _L), jnp.float32),
            pltpu.VMEM((_C, _L), jnp.float32),
        ],
    )
    return run(mem, emb_table, gamma, beta)


@jax.jit
def kernel(mem, emb_table, gamma, beta):
    tc_out = _tc_part(mem[:_TCB], emb_table, gamma, beta)
    sc_out = _sc_part(mem[_TCB:], emb_table, gamma, beta)
    return jnp.concatenate([tc_out, sc_out], axis=0)


# speedup vs baseline: 1.9674x; 1.2335x over previous
import functools

import jax
import jax.numpy as jnp
from jax import lax
from jax.experimental import pallas as pl
from jax.experimental.pallas import tpu as pltpu
from jax.experimental.pallas import tpu_sc as plsc

MEM_LENGTH = 8192
HIDDEN = 768
BATCH = 4

_BL = 512

_NC = 2
_NS = 16
_L = 16
_NW = _NC * _NS
_LPW = MEM_LENGTH // _NW
_C = 16
_NJ = HIDDEN // _L
_JB = 8

_GATHER_DNUMS = lax.GatherDimensionNumbers(
    offset_dims=(), collapsed_slice_dims=(0,), start_index_map=(0,))


def _lane_shuffle(v, perm):
    return lax.gather(v, perm[:, None], _GATHER_DNUMS, slice_sizes=(1,),
                      mode=lax.GatherScatterMode.PROMISE_IN_BOUNDS)


def _allsum_vec(v):
    idx = lax.iota(jnp.int32, 16)
    for sh in (8, 4, 2, 1):
        perm = lax.bitwise_xor(idx, jnp.int32(sh))
        v = v + _lane_shuffle(v, perm)
    return v


def _rsqrt_vec(x):
    i = lax.bitcast_convert_type(x, jnp.int32)
    i = jnp.int32(0x5F3759DF) - lax.shift_right_logical(i, 1)
    y = lax.bitcast_convert_type(i, jnp.float32)
    for _ in range(3):
        y = y * (1.5 - 0.5 * x * y * y)
    return y


_TCB = 3


def _ln_body(mem_ref, emb_ref, gamma_ref, beta_ref, out_ref):
    x = mem_ref[0] + emb_ref[...]
    mean = jnp.mean(x, axis=-1, keepdims=True)
    xc = x - mean
    var = jnp.mean(xc * xc, axis=-1, keepdims=True)
    inv = jax.lax.rsqrt(var + 1e-5)
    out_ref[0] = xc * inv * gamma_ref[...] + beta_ref[...]


def _tc_part(mem, emb_table, gamma, beta, nb):
    grid = (nb, MEM_LENGTH // _BL)
    return pl.pallas_call(
        _ln_body,
        grid=grid,
        in_specs=[
            pl.BlockSpec((1, _BL, HIDDEN), lambda b, i: (b, i, 0)),
            pl.BlockSpec((_BL, HIDDEN), lambda b, i: (i, 0)),
            pl.BlockSpec((HIDDEN,), lambda b, i: (0,)),
            pl.BlockSpec((HIDDEN,), lambda b, i: (0,)),
        ],
        out_specs=pl.BlockSpec((1, _BL, HIDDEN), lambda b, i: (b, i, 0)),
        out_shape=jax.ShapeDtypeStruct((nb, MEM_LENGTH, HIDDEN), jnp.float32),
    )(mem, emb_table, gamma, beta)


def _sc_body(b_lo, b_hi, mem, emb, gamma, beta, out, gamma_v, beta_v, emb_v, x_v, rs_v, ms_v):
    cid = lax.axis_index("c")
    sid = lax.axis_index("s")
    wid = sid * _NC + cid
    pltpu.sync_copy(gamma, gamma_v)
    pltpu.sync_copy(beta, beta_v)
    base = wid * _LPW

    zero = jnp.zeros((_L,), jnp.float32)

    def chunk(i, carry):
        l0 = base + i * _C
        pltpu.sync_copy(emb.at[pl.ds(l0, _C)], emb_v)
        for b in range(b_lo, b_hi):
            pltpu.sync_copy(mem.at[b, pl.ds(l0, _C)], x_v)

            def pass1(r, rc):
                s = zero
                sq = zero
                for j in range(_NJ):
                    sl = pl.ds(j * _L, _L)
                    v = x_v[r, sl] + emb_v[r, sl]
                    x_v[r, sl] = v
                    s = s + v
                    sq = sq + v * v
                mean = _allsum_vec(s) * (1.0 / HIDDEN)
                var = _allsum_vec(sq) * (1.0 / HIDDEN) - mean * mean
                rstd = _rsqrt_vec(var + 1e-5)
                rs_v[r, :] = rstd
                ms_v[r, :] = mean * rstd
                return rc

            lax.fori_loop(0, _C, pass1, 0)

            for jb in range(_NJ // _JB):
                gs = [gamma_v[pl.ds((jb * _JB + k) * _L, _L)] for k in range(_JB)]
                bs = [beta_v[pl.ds((jb * _JB + k) * _L, _L)] for k in range(_JB)]

                def pass2(r, rc, jb=jb, gs=gs, bs=bs):
                    rs = rs_v[r, :]
                    ms = ms_v[r, :]
                    for k in range(_JB):
                        sl = pl.ds((jb * _JB + k) * _L, _L)
                        v = x_v[r, sl]
                        x_v[r, sl] = (v * rs - ms) * gs[k] + bs[k]
                    return rc

                lax.fori_loop(0, _C, pass2, 0)

            pltpu.sync_copy(x_v, out.at[b - b_lo, pl.ds(l0, _C)])
        return carry

    lax.fori_loop(0, _LPW // _C, chunk, 0)


def _sc_part(mem, emb_table, gamma, beta, b_lo, b_hi):
    mesh = plsc.VectorSubcoreMesh(core_axis_name="c", subcore_axis_name="s")
    run = pl.kernel(
        functools.partial(_sc_body, b_lo, b_hi),
        mesh=mesh,
        out_type=jax.ShapeDtypeStruct((b_hi - b_lo, MEM_LENGTH, HIDDEN), jnp.float32),
        scratch_types=[
            pltpu.VMEM((HIDDEN,), jnp.float32),
            pltpu.VMEM((HIDDEN,), jnp.float32),
            pltpu.VMEM((_C, HIDDEN), jnp.float32),
            pltpu.VMEM((_C, HIDDEN), jnp.float32),
            pltpu.VMEM((_C, _L), jnp.float32),
            pltpu.VMEM((_C, _L), jnp.float32),
        ],
    )
    return run(mem, emb_table, gamma, beta)


@jax.jit
def kernel(mem, emb_table, gamma, beta):
    tc_out = _tc_part(mem, emb_table, gamma, beta, _TCB)
    sc_out = _sc_part(mem, emb_table, gamma, beta, _TCB, BATCH)
    return jnp.concatenate([tc_out, sc_out], axis=0)


# speedup vs baseline: 2.5609x; 1.3016x over previous
import functools

import jax
import jax.numpy as jnp
from jax import lax
from jax.experimental import pallas as pl
from jax.experimental.pallas import tpu as pltpu
from jax.experimental.pallas import tpu_sc as plsc

MEM_LENGTH = 8192
HIDDEN = 768
BATCH = 4

_BL = 512

_NC = 2
_NS = 16
_L = 16
_NW = _NC * _NS
_LPW = MEM_LENGTH // _NW
_C = 16
_NJ = HIDDEN // _L
_JB = 8

_GATHER_DNUMS = lax.GatherDimensionNumbers(
    offset_dims=(), collapsed_slice_dims=(0,), start_index_map=(0,))


def _lane_shuffle(v, perm):
    return lax.gather(v, perm[:, None], _GATHER_DNUMS, slice_sizes=(1,),
                      mode=lax.GatherScatterMode.PROMISE_IN_BOUNDS)


def _allsum_vec(v):
    idx = lax.iota(jnp.int32, 16)
    for sh in (8, 4, 2, 1):
        perm = lax.bitwise_xor(idx, jnp.int32(sh))
        v = v + _lane_shuffle(v, perm)
    return v


def _rsqrt_vec(x):
    i = lax.bitcast_convert_type(x, jnp.int32)
    i = jnp.int32(0x5F3759DF) - lax.shift_right_logical(i, 1)
    y = lax.bitcast_convert_type(i, jnp.float32)
    for _ in range(3):
        y = y * (1.5 - 0.5 * x * y * y)
    return y


_TCB = 3


def _ln_body(mem_ref, emb_ref, gamma_ref, beta_ref, out_ref):
    x = mem_ref[0] + emb_ref[...]
    mean = jnp.mean(x, axis=-1, keepdims=True)
    xc = x - mean
    var = jnp.mean(xc * xc, axis=-1, keepdims=True)
    inv = jax.lax.rsqrt(var + 1e-5)
    out_ref[0] = xc * inv * gamma_ref[...] + beta_ref[...]


def _tc_part(mem, emb_table, gamma, beta, nb, out_nb=None):
    grid = (nb, MEM_LENGTH // _BL)
    out_nb = nb if out_nb is None else out_nb
    return pl.pallas_call(
        _ln_body,
        grid=grid,
        in_specs=[
            pl.BlockSpec((1, _BL, HIDDEN), lambda b, i: (b, i, 0)),
            pl.BlockSpec((_BL, HIDDEN), lambda b, i: (i, 0)),
            pl.BlockSpec((HIDDEN,), lambda b, i: (0,)),
            pl.BlockSpec((HIDDEN,), lambda b, i: (0,)),
        ],
        out_specs=pl.BlockSpec((1, _BL, HIDDEN), lambda b, i: (b, i, 0)),
        out_shape=jax.ShapeDtypeStruct((out_nb, MEM_LENGTH, HIDDEN), jnp.float32),
    )(mem, emb_table, gamma, beta)


def _sc_body(b_lo, b_hi, mem, emb, gamma, beta, out, gamma_v, beta_v, emb_v, x_v, rs_v, ms_v):
    cid = lax.axis_index("c")
    sid = lax.axis_index("s")
    wid = sid * _NC + cid
    pltpu.sync_copy(gamma, gamma_v)
    pltpu.sync_copy(beta, beta_v)
    base = wid * _LPW

    zero = jnp.zeros((_L,), jnp.float32)

    def chunk(i, carry):
        l0 = base + i * _C
        pltpu.sync_copy(emb.at[pl.ds(l0, _C)], emb_v)
        for b in range(b_lo, b_hi):
            pltpu.sync_copy(mem.at[b, pl.ds(l0, _C)], x_v)

            def pass1(r, rc):
                s = zero
                sq = zero
                for j in range(_NJ):
                    sl = pl.ds(j * _L, _L)
                    v = x_v[r, sl] + emb_v[r, sl]
                    x_v[r, sl] = v
                    s = s + v
                    sq = sq + v * v
                mean = _allsum_vec(s) * (1.0 / HIDDEN)
                var = _allsum_vec(sq) * (1.0 / HIDDEN) - mean * mean
                rstd = _rsqrt_vec(var + 1e-5)
                rs_v[r, :] = rstd
                ms_v[r, :] = mean * rstd
                return rc

            lax.fori_loop(0, _C, pass1, 0)

            for jb in range(_NJ // _JB):
                gs = [gamma_v[pl.ds((jb * _JB + k) * _L, _L)] for k in range(_JB)]
                bs = [beta_v[pl.ds((jb * _JB + k) * _L, _L)] for k in range(_JB)]

                def pass2(r, rc, jb=jb, gs=gs, bs=bs):
                    rs = rs_v[r, :]
                    ms = ms_v[r, :]
                    for k in range(_JB):
                        sl = pl.ds((jb * _JB + k) * _L, _L)
                        v = x_v[r, sl]
                        x_v[r, sl] = (v * rs - ms) * gs[k] + bs[k]
                    return rc

                lax.fori_loop(0, _C, pass2, 0)

            pltpu.sync_copy(x_v, out.at[b - b_lo, pl.ds(l0, _C)])
        return carry

    lax.fori_loop(0, _LPW // _C, chunk, 0)


def _sc_part(mem, emb_table, gamma, beta, b_lo, b_hi):
    mesh = plsc.VectorSubcoreMesh(core_axis_name="c", subcore_axis_name="s")
    run = pl.kernel(
        functools.partial(_sc_body, b_lo, b_hi),
        mesh=mesh,
        out_type=jax.ShapeDtypeStruct((b_hi - b_lo, MEM_LENGTH, HIDDEN), jnp.float32),
        scratch_types=[
            pltpu.VMEM((HIDDEN,), jnp.float32),
            pltpu.VMEM((HIDDEN,), jnp.float32),
            pltpu.VMEM((_C, HIDDEN), jnp.float32),
            pltpu.VMEM((_C, HIDDEN), jnp.float32),
            pltpu.VMEM((_C, _L), jnp.float32),
            pltpu.VMEM((_C, _L), jnp.float32),
        ],
    )
    return run(mem, emb_table, gamma, beta)


@jax.jit
def kernel(mem, emb_table, gamma, beta):
    tc_out = _tc_part(mem, emb_table, gamma, beta, _TCB, out_nb=BATCH)
    sc_out = _sc_part(mem, emb_table, gamma, beta, _TCB, BATCH)
    tc_out, sc_out = lax.optimization_barrier((tc_out, sc_out))
    return lax.dynamic_update_slice(tc_out, sc_out, (_TCB, 0, 0))
